# Initial kernel scaffold; baseline (speedup 1.0000x reference)
import jax
import jax.numpy as jnp
from jax.experimental import pallas as pl


def kernel(X, E, epoch, Y, idx_train, Ws, W2s, W3s, a_s, a2_s, ctxs):
    raise NotImplementedError("write your pallas kernel here")


# speedup vs baseline: 1.9627x; 1.9627x over previous
import jax
import jax.numpy as jnp
from jax.experimental import pallas as pl
from jax.experimental.pallas import tpu as pltpu

_N = 10000
_M = 2000
_IN = 256
_HID = 64
_HEADS = 4

_NPAD = 10240
_NB = 512
_MB = 400
_NT = _NPAD // _NB
_MT = _M // _MB
_THRESH = 0.01
_SLOPE = 0.2


def _prep_body(x_ref, ws_ref, w2s_ref, a_ref, a2_ref, ctx_ref,
               xt_ref, g_ref, p_ref):
    x = x_ref[...]
    xts, gs, prows = [], [], []
    for h in range(_HEADS):
        xt = jnp.dot(x, ws_ref[h], preferred_element_type=jnp.float32)
        x4 = jnp.dot(x, w2s_ref[h], preferred_element_type=jnp.float32)
        c = jnp.sum(ctx_ref[h:h + 1, :] * a_ref[h:h + 1, :_HID])
        s = jax.lax.dot_general(x4, a_ref[h:h + 1, _HID:],
                                (((1,), (1,)), ((), ())),
                                preferred_element_type=jnp.float32)
        s = c + s
        s = jnp.maximum(s, _SLOPE * s)
        gs.append(jnp.exp(s))
        prow = jax.lax.dot_general(a2_ref[h:h + 1, :_HID], x4,
                                   (((1,), (1,)), ((), ())),
                                   preferred_element_type=jnp.float32)
        prows.append(prow)
        xts.append(xt)
    xt_ref[...] = jnp.concatenate(xts, axis=1)
    g_ref[...] = jnp.concatenate(gs, axis=1)
    p_ref[...] = jnp.concatenate(
        prows + [jnp.zeros((8 - _HEADS, _NB), jnp.float32)], axis=0)


def _stage1_body(e_ref, xt_ref, g_ref, w3_ref, a2_ref,
                 edge_ref, q_ref, acc_ref):
    j = pl.program_id(1)

    @pl.when(j == 0)
    def _():
        acc_ref[...] = jnp.zeros_like(acc_ref)

    col = j * _NB + jax.lax.broadcasted_iota(jnp.int32, (_MB, _NB), 1)
    valid = (e_ref[...] < _THRESH) & (col < _N)
    mask = jnp.where(valid, 1.0, 0.0)
    g = g_ref[...]
    parts = []
    for h in range(_HEADS):
        parts.append(xt_ref[:, h * _HID:(h + 1) * _HID] * g[:, h:h + 1])
        parts.append(g[:, h:h + 1])
    v = jnp.concatenate(parts, axis=1)
    acc_ref[...] += jnp.dot(mask, v, preferred_element_type=jnp.float32)

    @pl.when(j == _NT - 1)
    def _():
        acc = acc_ref[...]
        edges, qs = [], []
        for h in range(_HEADS):
            base = h * (_HID + 1)
            num = acc[:, base:base + _HID]
            den = acc[:, base + _HID:base + _HID + 1]
            den = jnp.where(den > 0, den, 1.0)
            edge = num / den
            e4 = jnp.dot(edge, w3_ref[h], preferred_element_type=jnp.float32)
            q = jax.lax.dot_general(e4, a2_ref[h:h + 1, _HID:],
                                    (((1,), (1,)), ((), ())),
                                    preferred_element_type=jnp.float32)
            edges.append(edge)
            qs.append(q)
        edge_ref[...] = jnp.concatenate(edges, axis=1)
        q_ref[...] = jnp.concatenate(qs, axis=1)


def _stage2_body(e_ref, edge_ref, q_ref, p_ref, out_ref, acc_ref, esum_ref):
    i = pl.program_id(0)
    j = pl.program_id(1)

    @pl.when(j == 0)
    def _():
        acc_ref[...] = jnp.zeros_like(acc_ref)
        esum_ref[...] = jnp.zeros_like(esum_ref)

    col = i * _NB + jax.lax.broadcasted_iota(jnp.int32, (_MB, _NB), 1)
    valid = (e_ref[...] < _THRESH) & (col < _N)
    edge = edge_ref[...]
    esum_ref[...] += jnp.sum(edge, axis=0, keepdims=True)
    exts = []
    for h in range(_HEADS):
        exts.append(edge[:, h * _HID:(h + 1) * _HID])
        exts.append(jnp.ones((_MB, 1), jnp.float32))
        exts.append(jnp.zeros((_MB, 128 - _HID - 1), jnp.float32))
    ext = jnp.concatenate(exts, axis=1)
    for h in range(_HEADS):
        t = q_ref[:, h:h + 1] + p_ref[h:h + 1, :]
        s = jnp.maximum(t, _SLOPE * t)
        w = jnp.where(valid, jnp.exp(s), 0.0)
        acc_ref[:, 128 * h:128 * (h + 1)] += jax.lax.dot_general(
            w, ext[:, 128 * h:128 * (h + 1)], (((0,), (0,)), ((), ())),
            preferred_element_type=jnp.float32)

    @pl.when(j == _MT - 1)
    def _():
        acc = acc_ref[...]
        esum = esum_ref[...]
        outs = []
        for h in range(_HEADS):
            num = acc[:, 128 * h:128 * h + _HID]
            den = acc[:, 128 * h + _HID:128 * h + _HID + 1]
            mean = esum[0:1, h * _HID:(h + 1) * _HID] * (1.0 / _M)
            node = jnp.where(den > 0, num / jnp.where(den > 0, den, 1.0), mean)
            outs.append(jnp.where(node > 0, node, jnp.exp(node) - 1.0))
        out_ref[...] = jnp.concatenate(outs, axis=1)


def kernel(X, E, epoch, Y, idx_train, Ws, W2s, W3s, a_s, a2_s, ctxs):
    X = X.astype(jnp.float32)
    E = E.astype(jnp.float32)
    Xp = jnp.pad(X, ((0, _NPAD - _N), (0, 0)))

    xt, g, p8 = pl.pallas_call(
        _prep_body,
        grid=(_NT,),
        in_specs=[
            pl.BlockSpec((_NB, _IN), lambda i: (i, 0)),
            pl.BlockSpec((_HEADS, _IN, _HID), lambda i: (0, 0, 0)),
            pl.BlockSpec((_HEADS, _IN, _HID), lambda i: (0, 0, 0)),
            pl.BlockSpec((_HEADS, 2 * _HID), lambda i: (0, 0)),
            pl.BlockSpec((_HEADS, 2 * _HID), lambda i: (0, 0)),
            pl.BlockSpec((_HEADS, _HID), lambda i: (0, 0)),
        ],
        out_specs=[
            pl.BlockSpec((_NB, _HEADS * _HID), lambda i: (i, 0)),
            pl.BlockSpec((_NB, _HEADS), lambda i: (i, 0)),
            pl.BlockSpec((8, _NB), lambda i: (0, i)),
        ],
        out_shape=[
            jax.ShapeDtypeStruct((_NPAD, _HEADS * _HID), jnp.float32),
            jax.ShapeDtypeStruct((_NPAD, _HEADS), jnp.float32),
            jax.ShapeDtypeStruct((8, _NPAD), jnp.float32),
        ],
    )(Xp, Ws, W2s, a_s, a2_s, ctxs)

    edge, q = pl.pallas_call(
        _stage1_body,
        grid=(_MT, _NT),
        in_specs=[
            pl.BlockSpec((_MB, _NB), lambda i, j: (i, j)),
            pl.BlockSpec((_NB, _HEADS * _HID), lambda i, j: (j, 0)),
            pl.BlockSpec((_NB, _HEADS), lambda i, j: (j, 0)),
            pl.BlockSpec((_HEADS, _HID, _HID), lambda i, j: (0, 0, 0)),
            pl.BlockSpec((_HEADS, 2 * _HID), lambda i, j: (0, 0)),
        ],
        out_specs=[
            pl.BlockSpec((_MB, _HEADS * _HID), lambda i, j: (i, 0)),
            pl.BlockSpec((_MB, _HEADS), lambda i, j: (i, 0)),
        ],
        out_shape=[
            jax.ShapeDtypeStruct((_M, _HEADS * _HID), jnp.float32),
            jax.ShapeDtypeStruct((_M, _HEADS), jnp.float32),
        ],
        scratch_shapes=[pltpu.VMEM((_MB, _HEADS * (_HID + 1)), jnp.float32)],
        compiler_params=pltpu.CompilerParams(
            dimension_semantics=("arbitrary", "arbitrary")),
    )(E, xt, g, W3s, a2_s)

    node = pl.pallas_call(
        _stage2_body,
        grid=(_NT, _MT),
        in_specs=[
            pl.BlockSpec((_MB, _NB), lambda i, j: (j, i)),
            pl.BlockSpec((_MB, _HEADS * _HID), lambda i, j: (j, 0)),
            pl.BlockSpec((_MB, _HEADS), lambda i, j: (j, 0)),
            pl.BlockSpec((8, _NB), lambda i, j: (0, i)),
        ],
        out_specs=pl.BlockSpec((_NB, _HEADS * _HID), lambda i, j: (i, 0)),
        out_shape=jax.ShapeDtypeStruct((_NPAD, _HEADS * _HID), jnp.float32),
        scratch_shapes=[pltpu.VMEM((_NB, 128 * _HEADS), jnp.float32),
                        pltpu.VMEM((1, _HEADS * _HID), jnp.float32)],
        compiler_params=pltpu.CompilerParams(
            dimension_semantics=("arbitrary", "arbitrary")),
    )(E, edge, q, p8)

    return node[:_N]


# speedup vs baseline: 2.2573x; 1.1501x over previous
import jax
import jax.numpy as jnp
from jax.experimental import pallas as pl
from jax.experimental.pallas import tpu as pltpu

_N = 10000
_M = 2000
_IN = 256
_HID = 64
_HEADS = 4

_NB = 512
_NPAD = 10240
_MB = 400
_NT = _NPAD // _NB
_MT = _M // _MB
_THRESH = 0.01
_SLOPE = 0.2
_LOG2E = 1.4426950408889634
_VW = _HEADS * (_HID + 1)
_EXTW = 128 * _HEADS


def _thresh_row(tile_idx, mb, nb):
    col = tile_idx * nb + jax.lax.broadcasted_iota(jnp.int32, (1, nb), 1)
    thr = jnp.where(col < _N, jnp.float32(_THRESH), jnp.float32(-1.0))
    return jnp.broadcast_to(thr, (mb, nb))


def _prep_body(x_ref, ws_ref, w2s_ref, a_ref, a2_ref, ctx_ref, v_ref, p_ref):
    x = x_ref[...]
    parts, prows = [], []
    for h in range(_HEADS):
        xt = jnp.dot(x, ws_ref[h], preferred_element_type=jnp.float32)
        x4 = jnp.dot(x, w2s_ref[h], preferred_element_type=jnp.float32)
        c = jnp.sum(ctx_ref[h:h + 1, :] * a_ref[h:h + 1, :_HID])
        s = jax.lax.dot_general(x4, a_ref[h:h + 1, _HID:],
                                (((1,), (1,)), ((), ())),
                                preferred_element_type=jnp.float32)
        s = c + s
        s = jnp.maximum(s, _SLOPE * s)
        g = jnp.exp(s)
        parts.append(xt * g)
        parts.append(g)
        prow = jax.lax.dot_general(a2_ref[h:h + 1, :_HID], x4,
                                   (((1,), (1,)), ((), ())),
                                   preferred_element_type=jnp.float32)
        prows.append(prow * _LOG2E)
    v_ref[...] = jnp.concatenate(parts, axis=1).astype(jnp.bfloat16)
    p_ref[...] = jnp.concatenate(
        prows + [jnp.zeros((8 - _HEADS, x.shape[0]), jnp.float32)], axis=0)


def _stage1_body(e_ref, v_ref, w3_ref, a2_ref,
                 ext_ref, q_ref, esum_ref, acc_ref):
    j = pl.program_id(1)

    @pl.when(j == 0)
    def _():
        acc_ref[...] = jnp.zeros_like(acc_ref)

    valid = e_ref[...] < _thresh_row(j, _MB, _NB)
    mask = jnp.where(valid, 1.0, 0.0).astype(jnp.bfloat16)
    acc_ref[...] += jnp.dot(mask, v_ref[...],
                            preferred_element_type=jnp.float32)

    @pl.when(j == _NT - 1)
    def _():
        acc = acc_ref[...]
        edges, exts, qs = [], [], []
        for h in range(_HEADS):
            base = h * (_HID + 1)
            num = acc[:, base:base + _HID]
            den = acc[:, base + _HID:base + _HID + 1]
            den = jnp.where(den > 0, den, 1.0)
            edge = num / den
            e4 = jnp.dot(edge, w3_ref[h], preferred_element_type=jnp.float32)
            q = jax.lax.dot_general(e4, a2_ref[h:h + 1, _HID:],
                                    (((1,), (1,)), ((), ())),
                                    preferred_element_type=jnp.float32)
            edges.append(edge)
            qs.append(q * _LOG2E)
            exts.append(edge.astype(jnp.bfloat16))
            exts.append(jnp.ones((_MB, 1), jnp.bfloat16))
            exts.append(jnp.zeros((_MB, 128 - _HID - 1), jnp.bfloat16))
        ext_ref[...] = jnp.concatenate(exts, axis=1)
        q_ref[...] = jnp.concatenate(qs, axis=1)
        esum = jnp.sum(jnp.concatenate(edges, axis=1), axis=0, keepdims=True)
        esum_ref[...] = esum.reshape(1, 1, _HEADS * _HID)


def _stage2_body(e_ref, ext_ref, q_ref, p_ref, esum_ref, out_ref, acc_ref):
    i = pl.program_id(0)
    j = pl.program_id(1)

    @pl.when(j == 0)
    def _():
        acc_ref[...] = jnp.zeros_like(acc_ref)

    valid = e_ref[...] < _thresh_row(i, _MB, _NB)
    ext = ext_ref[...]
    for h in range(_HEADS):
        t = q_ref[:, h:h + 1] + p_ref[h:h + 1, :]
        s = jnp.maximum(t, _SLOPE * t)
        w = jnp.where(valid, jnp.exp2(s), 0.0).astype(jnp.bfloat16)
        acc_ref[:, 128 * h:128 * (h + 1)] += jax.lax.dot_general(
            w, ext[:, 128 * h:128 * (h + 1)], (((0,), (0,)), ((), ())),
            preferred_element_type=jnp.float32)

    @pl.when(j == _MT - 1)
    def _():
        acc = acc_ref[...]
        esum = jnp.sum(esum_ref[...], axis=0)
        outs = []
        for h in range(_HEADS):
            num = acc[:, 128 * h:128 * h + _HID]
            den = acc[:, 128 * h + _HID:128 * h + _HID + 1]
            mean = esum[0:1, h * _HID:(h + 1) * _HID] * (1.0 / _M)
            node = jnp.where(den > 0, num / jnp.where(den > 0, den, 1.0), mean)
            outs.append(jnp.where(node > 0, node, jnp.exp(node) - 1.0))
        out_ref[...] = jnp.concatenate(outs, axis=1)


def kernel(X, E, epoch, Y, idx_train, Ws, W2s, W3s, a_s, a2_s, ctxs):
    X = X.astype(jnp.float32)
    E = E.astype(jnp.float32)
    Xp = jnp.pad(X, ((0, _NPAD - _N), (0, 0)))

    v, p8 = pl.pallas_call(
        _prep_body,
        grid=(_NT,),
        in_specs=[
            pl.BlockSpec((_NB, _IN), lambda i: (i, 0)),
            pl.BlockSpec((_HEADS, _IN, _HID), lambda i: (0, 0, 0)),
            pl.BlockSpec((_HEADS, _IN, _HID), lambda i: (0, 0, 0)),
            pl.BlockSpec((_HEADS, 2 * _HID), lambda i: (0, 0)),
            pl.BlockSpec((_HEADS, 2 * _HID), lambda i: (0, 0)),
            pl.BlockSpec((_HEADS, _HID), lambda i: (0, 0)),
        ],
        out_specs=[
            pl.BlockSpec((_NB, _VW), lambda i: (i, 0)),
            pl.BlockSpec((8, _NB), lambda i: (0, i)),
        ],
        out_shape=[
            jax.ShapeDtypeStruct((_NPAD, _VW), jnp.bfloat16),
            jax.ShapeDtypeStruct((8, _NPAD), jnp.float32),
        ],
    )(Xp, Ws, W2s, a_s, a2_s, ctxs)

    ext, q, esum = pl.pallas_call(
        _stage1_body,
        grid=(_MT, _NT),
        in_specs=[
            pl.BlockSpec((_MB, _NB), lambda i, j: (i, j)),
            pl.BlockSpec((_NB, _VW), lambda i, j: (j, 0)),
            pl.BlockSpec((_HEADS, _HID, _HID), lambda i, j: (0, 0, 0)),
            pl.BlockSpec((_HEADS, 2 * _HID), lambda i, j: (0, 0)),
        ],
        out_specs=[
            pl.BlockSpec((_MB, _EXTW), lambda i, j: (i, 0)),
            pl.BlockSpec((_MB, _HEADS), lambda i, j: (i, 0)),
            pl.BlockSpec((1, 1, _HEADS * _HID), lambda i, j: (i, 0, 0)),
        ],
        out_shape=[
            jax.ShapeDtypeStruct((_M, _EXTW), jnp.bfloat16),
            jax.ShapeDtypeStruct((_M, _HEADS), jnp.float32),
            jax.ShapeDtypeStruct((_MT, 1, _HEADS * _HID), jnp.float32),
        ],
        scratch_shapes=[pltpu.VMEM((_MB, _VW), jnp.float32)],
        compiler_params=pltpu.CompilerParams(
            dimension_semantics=("arbitrary", "arbitrary")),
    )(E, v, W3s, a2_s)

    node = pl.pallas_call(
        _stage2_body,
        grid=(_NT, _MT),
        in_specs=[
            pl.BlockSpec((_MB, _NB), lambda i, j: (j, i)),
            pl.BlockSpec((_MB, _EXTW), lambda i, j: (j, 0)),
            pl.BlockSpec((_MB, _HEADS), lambda i, j: (j, 0)),
            pl.BlockSpec((8, _NB), lambda i, j: (0, i)),
            pl.BlockSpec((_MT, 1, _HEADS * _HID), lambda i, j: (0, 0, 0)),
        ],
        out_specs=pl.BlockSpec((_NB, _HEADS * _HID), lambda i, j: (i, 0)),
        out_shape=jax.ShapeDtypeStruct((_NPAD, _HEADS * _HID), jnp.float32),
        scratch_shapes=[pltpu.VMEM((_NB, _EXTW), jnp.float32)],
        compiler_params=pltpu.CompilerParams(
            dimension_semantics=("arbitrary", "arbitrary")),
    )(E, ext, q, p8, esum)

    return node[:_N]
